# initial kernel scaffold (unmeasured)
import jax
import jax.numpy as jnp
from jax import lax
from jax.experimental import pallas as pl
from jax.experimental.pallas import tpu as pltpu

N_DEV = 4
B = 2
SQ_SHARD = 512
SKV = 512
HQ_SHARD = 8
DH = 64
D_MODEL = 768
BLK = 64


def kernel(x, Wq, K_ext, V_ext, Wo):
    def body(x_ref, wq_ref, k_ref, v_ref, wo_ref, out_ref,
             wq_buf, wo_buf, wq_send, wq_recv, wo_send, wo_recv):
        my = lax.axis_index("i")
        left = (my + N_DEV - 1) % N_DEV
        right = (my + 1) % N_DEV

        barrier = pltpu.get_barrier_semaphore()
        for nbr in (left, right):
            pl.semaphore_signal(
                barrier, inc=1,
                device_id=(nbr,), device_id_type=pl.DeviceIdType.MESH,
            )
        pl.semaphore_wait(barrier, 2)

        qi = lax.broadcasted_iota(jnp.int32, (SQ_SHARD, SKV), 0) + my * SQ_SHARD
        ki = lax.broadcasted_iota(jnp.int32, (SQ_SHARD, SKV), 1)
        qb = qi // BLK
        kb = ki // BLK
        keep = (qb == kb) | ((qb % 4) == (kb % 4))
        neg = jnp.where(keep, 0.0, -1e9).astype(jnp.float32)

        def compute_group(g, wq_g, wo_g, accumulate):
            h0 = g * HQ_SHARD
            for b in range(B):
                xb = x_ref[b]
                q = lax.dot(xb, wq_g, preferred_element_type=jnp.float32)
                q3 = (q * 0.125).reshape(SQ_SHARD, HQ_SHARD, DH)
                kg = k_ref[b, :, pl.ds(h0, HQ_SHARD), :]
                vg = v_ref[b, :, pl.ds(h0, HQ_SHARD), :]
                s = lax.dot_general(
                    q3, kg, (((2,), (2,)), ((1,), (1,))),
                    preferred_element_type=jnp.float32,
                )
                s = s + neg[None, :, :]
                m = jnp.max(s, axis=-1, keepdims=True)
                w = jnp.exp(s - m)
                w = w / jnp.sum(w, axis=-1, keepdims=True)
                ctx = lax.dot_general(
                    w, vg, (((2,), (0,)), ((0,), (1,))),
                    preferred_element_type=jnp.float32,
                )
                ctx2 = jnp.transpose(ctx, (1, 0, 2)).reshape(
                    SQ_SHARD, HQ_SHARD * DH)
                part = lax.dot(ctx2, wo_g, preferred_element_type=jnp.float32)
                if accumulate:
                    out_ref[b, :, :] = out_ref[b, :, :] + part
                else:
                    out_ref[b, :, :] = part

        def make_pair(h):
            src_wq = wq_ref if h == 1 else wq_buf.at[h - 2]
            src_wo = wo_ref if h == 1 else wo_buf.at[h - 2]
            r_wq = pltpu.make_async_remote_copy(
                src_ref=src_wq, dst_ref=wq_buf.at[h - 1],
                send_sem=wq_send.at[h - 1], recv_sem=wq_recv.at[h - 1],
                device_id=(right,), device_id_type=pl.DeviceIdType.MESH,
            )
            r_wo = pltpu.make_async_remote_copy(
                src_ref=src_wo, dst_ref=wo_buf.at[h - 1],
                send_sem=wo_send.at[h - 1], recv_sem=wo_recv.at[h - 1],
                device_id=(right,), device_id_type=pl.DeviceIdType.MESH,
            )
            return r_wq, r_wo

        rdmas = {1: make_pair(1)}
        rdmas[1][0].start()
        rdmas[1][1].start()

        compute_group(my, wq_ref[...], wo_ref[...], accumulate=False)

        for h in range(1, N_DEV):
            r_wq, r_wo = rdmas[h]
            r_wq.wait_recv()
            r_wo.wait_recv()
            if h + 1 < N_DEV:
                rdmas[h + 1] = make_pair(h + 1)
                rdmas[h + 1][0].start()
                rdmas[h + 1][1].start()
            g = (my + (N_DEV - h)) % N_DEV
            compute_group(g, wq_buf[h - 1], wo_buf[h - 1], accumulate=True)

        for h in range(1, N_DEV):
            rdmas[h][0].wait_send()
            rdmas[h][1].wait_send()

    out_shape = jax.ShapeDtypeStruct((B, SQ_SHARD, D_MODEL), jnp.float32)
    return pl.pallas_call(
        body,
        out_shape=out_shape,
        in_specs=[pl.BlockSpec(memory_space=pltpu.VMEM)] * 5,
        out_specs=pl.BlockSpec(memory_space=pltpu.VMEM),
        scratch_shapes=[
            pltpu.VMEM((N_DEV - 1, D_MODEL, HQ_SHARD * DH), jnp.float32),
            pltpu.VMEM((N_DEV - 1, HQ_SHARD * DH, D_MODEL), jnp.float32),
            pltpu.SemaphoreType.DMA((N_DEV - 1,)),
            pltpu.SemaphoreType.DMA((N_DEV - 1,)),
            pltpu.SemaphoreType.DMA((N_DEV - 1,)),
            pltpu.SemaphoreType.DMA((N_DEV - 1,)),
        ],
        compiler_params=pltpu.CompilerParams(collective_id=0),
    )(x, Wq, K_ext, V_ext, Wo)


# baseline (device time: 152686 ns/iter reference)
import jax
import jax.numpy as jnp
from jax import lax
from jax.experimental import pallas as pl
from jax.experimental.pallas import tpu as pltpu

N_DEV = 4
B = 2
SQ_SHARD = 512
SKV = 512
HQ_SHARD = 8
DH = 64
D_MODEL = 768
BLK = 64


def kernel(x, Wq, K_ext, V_ext, Wo):
    def body(x_ref, wq_ref, k_ref, v_ref, wo_ref, out_ref,
             wq_buf, wo_buf, wq_send, wq_recv, wo_send, wo_recv):
        my = lax.axis_index("i")
        left = (my + N_DEV - 1) % N_DEV
        right = (my + 1) % N_DEV

        barrier = pltpu.get_barrier_semaphore()
        for nbr in (left, right):
            pl.semaphore_signal(
                barrier, inc=1,
                device_id=(nbr,), device_id_type=pl.DeviceIdType.MESH,
            )
        pl.semaphore_wait(barrier, 2)

        qi = lax.broadcasted_iota(jnp.int32, (SQ_SHARD, SKV), 0) + my * SQ_SHARD
        ki = lax.broadcasted_iota(jnp.int32, (SQ_SHARD, SKV), 1)
        qb = qi // BLK
        kb = ki // BLK
        keep = (qb == kb) | ((qb % 4) == (kb % 4))
        neg = jnp.where(keep, 0.0, -1e9).astype(jnp.float32)

        def compute_group(g, wq_g, wo_g, accumulate):
            c0 = g * (HQ_SHARD * DH)
            for b in range(B):
                xb = x_ref[b]
                q = lax.dot(xb, wq_g, preferred_element_type=jnp.float32)
                q = q * 0.125
                kg = k_ref[b, :, pl.ds(c0, HQ_SHARD * DH)]
                vg = v_ref[b, :, pl.ds(c0, HQ_SHARD * DH)]
                cols = []
                for h in range(HQ_SHARD):
                    sl = slice(h * DH, (h + 1) * DH)
                    s = lax.dot_general(
                        q[:, sl], kg[:, sl], (((1,), (1,)), ((), ())),
                        preferred_element_type=jnp.float32,
                    )
                    s = s + neg
                    m = jnp.max(s, axis=-1, keepdims=True)
                    w = jnp.exp(s - m)
                    w = w / jnp.sum(w, axis=-1, keepdims=True)
                    cols.append(
                        lax.dot(w, vg[:, sl],
                                preferred_element_type=jnp.float32))
                ctx = jnp.concatenate(cols, axis=1)
                part = lax.dot(ctx, wo_g, preferred_element_type=jnp.float32)
                if accumulate:
                    out_ref[b, :, :] = out_ref[b, :, :] + part
                else:
                    out_ref[b, :, :] = part

        def make_pair(h):
            src_wq = wq_ref if h == 1 else wq_buf.at[h - 2]
            src_wo = wo_ref if h == 1 else wo_buf.at[h - 2]
            r_wq = pltpu.make_async_remote_copy(
                src_ref=src_wq, dst_ref=wq_buf.at[h - 1],
                send_sem=wq_send.at[h - 1], recv_sem=wq_recv.at[h - 1],
                device_id=(right,), device_id_type=pl.DeviceIdType.MESH,
            )
            r_wo = pltpu.make_async_remote_copy(
                src_ref=src_wo, dst_ref=wo_buf.at[h - 1],
                send_sem=wo_send.at[h - 1], recv_sem=wo_recv.at[h - 1],
                device_id=(right,), device_id_type=pl.DeviceIdType.MESH,
            )
            return r_wq, r_wo

        rdmas = {1: make_pair(1)}
        rdmas[1][0].start()
        rdmas[1][1].start()

        compute_group(my, wq_ref[...], wo_ref[...], accumulate=False)

        for h in range(1, N_DEV):
            r_wq, r_wo = rdmas[h]
            r_wq.wait_recv()
            r_wo.wait_recv()
            if h + 1 < N_DEV:
                rdmas[h + 1] = make_pair(h + 1)
                rdmas[h + 1][0].start()
                rdmas[h + 1][1].start()
            g = (my + (N_DEV - h)) % N_DEV
            compute_group(g, wq_buf[h - 1], wo_buf[h - 1], accumulate=True)

        for h in range(1, N_DEV):
            rdmas[h][0].wait_send()
            rdmas[h][1].wait_send()

    K_flat = K_ext.reshape(B, SKV, 32 * DH)
    V_flat = V_ext.reshape(B, SKV, 32 * DH)

    out_shape = jax.ShapeDtypeStruct((B, SQ_SHARD, D_MODEL), jnp.float32)
    return pl.pallas_call(
        body,
        out_shape=out_shape,
        in_specs=[pl.BlockSpec(memory_space=pltpu.VMEM)] * 5,
        out_specs=pl.BlockSpec(memory_space=pltpu.VMEM),
        scratch_shapes=[
            pltpu.VMEM((N_DEV - 1, D_MODEL, HQ_SHARD * DH), jnp.float32),
            pltpu.VMEM((N_DEV - 1, HQ_SHARD * DH, D_MODEL), jnp.float32),
            pltpu.SemaphoreType.DMA((N_DEV - 1,)),
            pltpu.SemaphoreType.DMA((N_DEV - 1,)),
            pltpu.SemaphoreType.DMA((N_DEV - 1,)),
            pltpu.SemaphoreType.DMA((N_DEV - 1,)),
        ],
        compiler_params=pltpu.CompilerParams(
            collective_id=0,
            vmem_limit_bytes=100 * 1024 * 1024,
        ),
    )(x, Wq, K_flat, V_flat, Wo)
